# Initial kernel scaffold; baseline (speedup 1.0000x reference)
import jax
import jax.numpy as jnp
from jax.experimental import pallas as pl


def kernel(ufea, vfea, UV_adj, VU_adj, params):
    raise NotImplementedError("write your pallas kernel here")


# speedup vs baseline: 1.5054x; 1.5054x over previous
import functools

import jax
import jax.numpy as jnp
from jax.experimental import pallas as pl
from jax.experimental.pallas import tpu as pltpu

N = 10000
F = 128
ALPHA = 0.3

_BM_BIG = 200
_BM_SMALL = 1000


def _leaky(x):
    return jnp.where(x >= 0, x, ALPHA * x)


def _proj_body(x_ref, w_ref, o_ref):
    acc = jnp.dot(x_ref[...].astype(jnp.bfloat16), w_ref[...],
                  preferred_element_type=jnp.float32)
    o_ref[...] = acc.astype(jnp.bfloat16)


def _proj(x, w_bf):
    w = w_bf.shape[1]
    return pl.pallas_call(
        _proj_body,
        grid=(N // _BM_SMALL,),
        in_specs=[pl.BlockSpec((_BM_SMALL, F), lambda i: (i, 0)),
                  pl.BlockSpec((F, w), lambda i: (0, 0))],
        out_specs=pl.BlockSpec((_BM_SMALL, w), lambda i: (i, 0)),
        out_shape=jax.ShapeDtypeStruct((N, w), jnp.bfloat16),
    )(x, w_bf)


def _lin2_body(x1_ref, x2_ref, w1_ref, w2_ref, b_ref, o_ref, *, relu):
    acc = jnp.dot(x1_ref[...].astype(jnp.bfloat16), w1_ref[...],
                  preferred_element_type=jnp.float32)
    acc = acc + jnp.dot(x2_ref[...].astype(jnp.bfloat16), w2_ref[...],
                        preferred_element_type=jnp.float32)
    acc = acc + b_ref[...]
    if relu:
        acc = jnp.maximum(acc, 0.0)
    o_ref[...] = acc


def _lin2(x1, x2, w1_bf, w2_bf, b, relu):
    return pl.pallas_call(
        functools.partial(_lin2_body, relu=relu),
        grid=(N // _BM_SMALL,),
        in_specs=[pl.BlockSpec((_BM_SMALL, F), lambda i: (i, 0)),
                  pl.BlockSpec((_BM_SMALL, F), lambda i: (i, 0)),
                  pl.BlockSpec((F, F), lambda i: (0, 0)),
                  pl.BlockSpec((F, F), lambda i: (0, 0)),
                  pl.BlockSpec((1, F), lambda i: (0, 0))],
        out_specs=pl.BlockSpec((_BM_SMALL, F), lambda i: (i, 0)),
        out_shape=jax.ShapeDtypeStruct((N, F), jnp.float32),
    )(x1, x2, w1_bf, w2_bf, b)


def _adj_body(adj_ref, rhs_ref, b_ref, *o_refs):
    a = adj_ref[...].astype(jnp.bfloat16)
    acc = jnp.dot(a, rhs_ref[...], preferred_element_type=jnp.float32)
    acc = _leaky(acc + b_ref[...])
    for j, o in enumerate(o_refs):
        o[...] = acc[:, j * F:(j + 1) * F]


def _adj_pass(adj, rhs_bf, bias, n_out):
    w = F * n_out
    outs = pl.pallas_call(
        _adj_body,
        grid=(N // _BM_BIG,),
        in_specs=[pl.BlockSpec((_BM_BIG, N), lambda i: (i, 0)),
                  pl.BlockSpec((N, w), lambda i: (0, 0)),
                  pl.BlockSpec((1, w), lambda i: (0, 0))],
        out_specs=[pl.BlockSpec((_BM_BIG, F), lambda i: (i, 0))] * n_out,
        out_shape=[jax.ShapeDtypeStruct((N, F), jnp.float32)] * n_out,
        compiler_params=pltpu.CompilerParams(
            dimension_semantics=("arbitrary",)),
    )(adj, rhs_bf, bias)
    return outs


def kernel(ufea, vfea, UV_adj, VU_adj, params):
    p = params
    bf = lambda a: a.astype(jnp.bfloat16)

    def wcat(*names):
        return bf(jnp.concatenate([p[n] for n in names], axis=1))

    def bcat(*names):
        return jnp.concatenate([p[n] for n in names])[None, :]

    rA = _proj(ufea, bf(p['l0_gc1_W']))
    (uh1,) = _adj_pass(VU_adj, rA, p['l0_gc1_b'][None, :], 1)

    rB = jnp.concatenate([_proj(vfea, bf(p['l0_gc2_W'])),
                          _proj(uh1, bf(p['l0_gc3_W']))], axis=1)
    ih1, uh2 = _adj_pass(UV_adj, rB, bcat('l0_gc2_b', 'l0_gc3_b'), 2)

    u = _lin2(uh2, ufea, bf(p['l0_uu_W'][:F]), bf(p['l0_uu_W'][F:]),
              p['l0_uu_b'][None, :], relu=True)

    rC = jnp.concatenate([_proj(ih1, bf(p['l0_gc4_W'])),
                          _proj(u, bf(p['ll_gc1_W']))], axis=1)
    ih2, uhh = _adj_pass(VU_adj, rC, bcat('l0_gc4_b', 'll_gc1_b'), 2)

    v = _lin2(ih2, vfea, bf(p['l0_iu_W'][:F]), bf(p['l0_iu_W'][F:]),
              p['l0_iu_b'][None, :], relu=True)

    rD = jnp.concatenate([_proj(v, bf(p['ll_gc2_W'])),
                          _proj(uhh, wcat('ll_gc3m_W', 'll_gc3s_W'))], axis=1)
    ihh, gmu, gsu = _adj_pass(UV_adj, rD,
                              bcat('ll_gc2_b', 'll_gc3m_b', 'll_gc3s_b'), 3)

    rE = _proj(ihh, wcat('ll_gc4m_W', 'll_gc4s_W'))
    gmi, gsi = _adj_pass(VU_adj, rE, bcat('ll_gc4m_b', 'll_gc4s_b'), 2)

    mean_u = _lin2(gmu, u, bf(p['ll_uum_W'][:F]), bf(p['ll_uum_W'][F:]),
                   p['ll_uum_b'][None, :], relu=False)
    logstd_u = _lin2(gsu, u, bf(p['ll_uus_W'][:F]), bf(p['ll_uus_W'][F:]),
                     p['ll_uus_b'][None, :], relu=False)
    mean_i = _lin2(gmi, v, bf(p['ll_ium_W'][:F]), bf(p['ll_ium_W'][F:]),
                   p['ll_ium_b'][None, :], relu=False)
    logstd_i = _lin2(gsi, v, bf(p['ll_ius_W'][:F]), bf(p['ll_ius_W'][F:]),
                     p['ll_ius_b'][None, :], relu=False)

    return (mean_u, mean_i, mean_u, mean_i, logstd_u, logstd_i)


# speedup vs baseline: 1.7631x; 1.1711x over previous
import jax
import jax.numpy as jnp
from jax.experimental import pallas as pl
from jax.experimental.pallas import tpu as pltpu

N = 10000
F = 128
ALPHA = 0.3

_BM = 200
_BM_SMALL = 1000


def _leaky(x):
    return jnp.where(x >= 0, x, ALPHA * x)


def _bf(x):
    return x.astype(jnp.bfloat16)


def _dot(a, b):
    return jnp.dot(a, b, preferred_element_type=jnp.float32)


def _proj_body(x_ref, w_ref, o_ref):
    o_ref[...] = _bf(_dot(_bf(x_ref[...]), w_ref[...]))


def _proj(x, w_bf):
    w = w_bf.shape[1]
    return pl.pallas_call(
        _proj_body,
        grid=(N // _BM_SMALL,),
        in_specs=[pl.BlockSpec((_BM_SMALL, F), lambda i: (i, 0)),
                  pl.BlockSpec((F, w), lambda i: (0, 0))],
        out_specs=pl.BlockSpec((_BM_SMALL, w), lambda i: (i, 0)),
        out_shape=jax.ShapeDtypeStruct((N, w), jnp.bfloat16),
    )(x, w_bf)


def _pass(body, adj, rhs, bias, row_ins, res_ins, out_w, out_dt):
    w = rhs.shape[1]
    in_specs = [pl.BlockSpec((_BM, N), lambda i: (i, 0)),
                pl.BlockSpec((N, w), lambda i: (0, 0)),
                pl.BlockSpec((1, w), lambda i: (0, 0))]
    for a in row_ins:
        in_specs.append(pl.BlockSpec((_BM, a.shape[1]), lambda i: (i, 0)))
    for a in res_ins:
        in_specs.append(pl.BlockSpec(
            tuple(a.shape), lambda i, n=len(a.shape): (0,) * n))
    out_specs = [pl.BlockSpec((_BM, ww), lambda i: (i, 0)) for ww in out_w]
    out_shape = [jax.ShapeDtypeStruct((N, ww), dt)
                 for ww, dt in zip(out_w, out_dt)]
    return pl.pallas_call(
        body,
        grid=(N // _BM,),
        in_specs=in_specs,
        out_specs=out_specs,
        out_shape=out_shape,
        compiler_params=pltpu.CompilerParams(
            dimension_semantics=("arbitrary",)),
    )(adj, rhs, bias, *row_ins, *res_ins)


def _gcn_tile(adj_ref, rhs_ref, b_ref):
    a = adj_ref[...]
    if a.dtype != jnp.bfloat16:
        a = _bf(a)
    return _leaky(_dot(a, rhs_ref[...]) + b_ref[...])


def _passA_body(adj_ref, rhs_ref, b_ref, vfea_ref, w2_ref, w3_ref, rB_ref):
    uh1 = _gcn_tile(adj_ref, rhs_ref, b_ref)
    rB_ref[...] = jnp.concatenate(
        [_bf(_dot(_bf(vfea_ref[...]), w2_ref[...])),
         _bf(_dot(_bf(uh1), w3_ref[...]))], axis=1)


def _passB_body(adj_ref, rhs_ref, b_ref, ufea_ref,
                w4_ref, wll1_ref, wuu1_ref, wuu2_ref, buu_ref,
                rC_ref, u_ref):
    t = _gcn_tile(adj_ref, rhs_ref, b_ref)
    ih1, uh2 = t[:, :F], t[:, F:]
    u = jnp.maximum(_dot(_bf(uh2), wuu1_ref[...])
                    + _dot(_bf(ufea_ref[...]), wuu2_ref[...])
                    + buu_ref[...], 0.0)
    u_ref[...] = u
    rC_ref[...] = jnp.concatenate(
        [_bf(_dot(_bf(ih1), w4_ref[...])),
         _bf(_dot(_bf(u), wll1_ref[...]))], axis=1)


def _passC_body(adj_ref, rhs_ref, b_ref, vfea_ref,
                wll2_ref, w3ms_ref, wiu1_ref, wiu2_ref, biu_ref,
                rD_ref, v_ref):
    t = _gcn_tile(adj_ref, rhs_ref, b_ref)
    ih2, uhh = t[:, :F], t[:, F:]
    v = jnp.maximum(_dot(_bf(ih2), wiu1_ref[...])
                    + _dot(_bf(vfea_ref[...]), wiu2_ref[...])
                    + biu_ref[...], 0.0)
    v_ref[...] = v
    rD_ref[...] = jnp.concatenate(
        [_bf(_dot(_bf(v), wll2_ref[...])),
         _bf(_dot(_bf(uhh), w3ms_ref[...]))], axis=1)


def _passD_body(adj_ref, rhs_ref, b_ref, u_ref,
                w4ms_ref, wum1_ref, wum2_ref, bum_ref,
                wus1_ref, wus2_ref, bus_ref,
                rE_ref, mu_ref, lu_ref):
    t = _gcn_tile(adj_ref, rhs_ref, b_ref)
    ihh, gmu, gsu = t[:, :F], t[:, F:2 * F], t[:, 2 * F:]
    rE_ref[...] = _bf(_dot(_bf(ihh), w4ms_ref[...]))
    ub = _bf(u_ref[...])
    mu_ref[...] = (_dot(_bf(gmu), wum1_ref[...]) + _dot(ub, wum2_ref[...])
                   + bum_ref[...])
    lu_ref[...] = (_dot(_bf(gsu), wus1_ref[...]) + _dot(ub, wus2_ref[...])
                   + bus_ref[...])


def _passE_body(adj_ref, rhs_ref, b_ref, v_ref,
                wim1_ref, wim2_ref, bim_ref,
                wis1_ref, wis2_ref, bis_ref,
                mi_ref, li_ref):
    t = _gcn_tile(adj_ref, rhs_ref, b_ref)
    gmi, gsi = t[:, :F], t[:, F:]
    vb = _bf(v_ref[...])
    mi_ref[...] = (_dot(_bf(gmi), wim1_ref[...]) + _dot(vb, wim2_ref[...])
                   + bim_ref[...])
    li_ref[...] = (_dot(_bf(gsi), wis1_ref[...]) + _dot(vb, wis2_ref[...])
                   + bis_ref[...])


def kernel(ufea, vfea, UV_adj, VU_adj, params):
    p = params

    def wcat(*names):
        return _bf(jnp.concatenate([p[n] for n in names], axis=1))

    def bcat(*names):
        return jnp.concatenate([p[n] for n in names])[None, :]

    rA = _proj(ufea, _bf(p['l0_gc1_W']))

    (rB,) = _pass(
        _passA_body, VU_adj, rA, p['l0_gc1_b'][None, :],
        [vfea], [_bf(p['l0_gc2_W']), _bf(p['l0_gc3_W'])],
        [2 * F], [jnp.bfloat16])

    rC, u = _pass(
        _passB_body, UV_adj, rB, bcat('l0_gc2_b', 'l0_gc3_b'),
        [ufea],
        [_bf(p['l0_gc4_W']), _bf(p['ll_gc1_W']),
         _bf(p['l0_uu_W'][:F]), _bf(p['l0_uu_W'][F:]), p['l0_uu_b'][None, :]],
        [2 * F, F], [jnp.bfloat16, jnp.float32])

    rD, v = _pass(
        _passC_body, VU_adj, rC, bcat('l0_gc4_b', 'll_gc1_b'),
        [vfea],
        [_bf(p['ll_gc2_W']), wcat('ll_gc3m_W', 'll_gc3s_W'),
         _bf(p['l0_iu_W'][:F]), _bf(p['l0_iu_W'][F:]), p['l0_iu_b'][None, :]],
        [3 * F, F], [jnp.bfloat16, jnp.float32])

    rE, mean_u, logstd_u = _pass(
        _passD_body, UV_adj, rD, bcat('ll_gc2_b', 'll_gc3m_b', 'll_gc3s_b'),
        [u],
        [wcat('ll_gc4m_W', 'll_gc4s_W'),
         _bf(p['ll_uum_W'][:F]), _bf(p['ll_uum_W'][F:]), p['ll_uum_b'][None, :],
         _bf(p['ll_uus_W'][:F]), _bf(p['ll_uus_W'][F:]), p['ll_uus_b'][None, :]],
        [2 * F, F, F], [jnp.bfloat16, jnp.float32, jnp.float32])

    mean_i, logstd_i = _pass(
        _passE_body, VU_adj, rE, bcat('ll_gc4m_b', 'll_gc4s_b'),
        [v],
        [_bf(p['ll_ium_W'][:F]), _bf(p['ll_ium_W'][F:]), p['ll_ium_b'][None, :],
         _bf(p['ll_ius_W'][:F]), _bf(p['ll_ius_W'][F:]), p['ll_ius_b'][None, :]],
        [F, F], [jnp.float32, jnp.float32])

    return (mean_u, mean_i, mean_u, mean_i, logstd_u, logstd_i)


# speedup vs baseline: 1.7814x; 1.0104x over previous
import jax
import jax.numpy as jnp
from jax.experimental import pallas as pl
from jax.experimental.pallas import tpu as pltpu

N = 10000
F = 128
ALPHA = 0.3

_BM = 200
_BM_SMALL = 1000


def _leaky(x):
    return jnp.where(x >= 0, x, ALPHA * x)


def _bf(x):
    return x.astype(jnp.bfloat16)


def _dot(a, b):
    return jnp.dot(a, b, preferred_element_type=jnp.float32)


def _proj_body(x_ref, w_ref, o_ref):
    o_ref[...] = _bf(_dot(_bf(x_ref[...]), w_ref[...]))


def _proj(x, w_bf):
    w = w_bf.shape[1]
    return pl.pallas_call(
        _proj_body,
        grid=(N // _BM_SMALL,),
        in_specs=[pl.BlockSpec((_BM_SMALL, F), lambda i: (i, 0)),
                  pl.BlockSpec((F, w), lambda i: (0, 0))],
        out_specs=pl.BlockSpec((_BM_SMALL, w), lambda i: (i, 0)),
        out_shape=jax.ShapeDtypeStruct((N, w), jnp.bfloat16),
    )(x, w_bf)


def _pass(body, adj, rhs, bias, row_ins, res_ins, out_w, out_dt):
    w = rhs.shape[1]
    in_specs = [pl.BlockSpec((_BM, N), lambda i: (i, 0)),
                pl.BlockSpec((N, w), lambda i: (0, 0)),
                pl.BlockSpec((1, w), lambda i: (0, 0))]
    for a in row_ins:
        in_specs.append(pl.BlockSpec((_BM, a.shape[1]), lambda i: (i, 0)))
    for a in res_ins:
        in_specs.append(pl.BlockSpec(
            tuple(a.shape), lambda i, n=len(a.shape): (0,) * n))
    out_specs = [pl.BlockSpec((_BM, ww), lambda i: (i, 0)) for ww in out_w]
    out_shape = [jax.ShapeDtypeStruct((N, ww), dt)
                 for ww, dt in zip(out_w, out_dt)]
    return pl.pallas_call(
        body,
        grid=(N // _BM,),
        in_specs=in_specs,
        out_specs=out_specs,
        out_shape=out_shape,
        compiler_params=pltpu.CompilerParams(
            dimension_semantics=("arbitrary",)),
    )(adj, rhs, bias, *row_ins, *res_ins)


def _gcn_tile(adj_ref, rhs_ref, b_ref):
    a = adj_ref[...]
    if a.dtype != jnp.bfloat16:
        a = _bf(a)
    return _leaky(_dot(a, rhs_ref[...]) + b_ref[...])


def _passA_body(adj_ref, rhs_ref, b_ref, vfea_ref, w2_ref, w3_ref,
                abf_ref, rB_ref):
    a = _bf(adj_ref[...])
    abf_ref[...] = a
    uh1 = _leaky(_dot(a, rhs_ref[...]) + b_ref[...])
    rB_ref[...] = jnp.concatenate(
        [_bf(_dot(_bf(vfea_ref[...]), w2_ref[...])),
         _bf(_dot(_bf(uh1), w3_ref[...]))], axis=1)


def _passB_body(adj_ref, rhs_ref, b_ref, ufea_ref,
                w4_ref, wll1_ref, wuu1_ref, wuu2_ref, buu_ref,
                rC_ref, u_ref):
    t = _gcn_tile(adj_ref, rhs_ref, b_ref)
    ih1, uh2 = t[:, :F], t[:, F:]
    u = jnp.maximum(_dot(_bf(uh2), wuu1_ref[...])
                    + _dot(_bf(ufea_ref[...]), wuu2_ref[...])
                    + buu_ref[...], 0.0)
    u_ref[...] = u
    rC_ref[...] = jnp.concatenate(
        [_bf(_dot(_bf(ih1), w4_ref[...])),
         _bf(_dot(_bf(u), wll1_ref[...]))], axis=1)


def _passC_body(adj_ref, rhs_ref, b_ref, vfea_ref,
                wll2_ref, w3ms_ref, wiu1_ref, wiu2_ref, biu_ref,
                rD_ref, v_ref):
    t = _gcn_tile(adj_ref, rhs_ref, b_ref)
    ih2, uhh = t[:, :F], t[:, F:]
    v = jnp.maximum(_dot(_bf(ih2), wiu1_ref[...])
                    + _dot(_bf(vfea_ref[...]), wiu2_ref[...])
                    + biu_ref[...], 0.0)
    v_ref[...] = v
    rD_ref[...] = jnp.concatenate(
        [_bf(_dot(_bf(v), wll2_ref[...])),
         _bf(_dot(_bf(uhh), w3ms_ref[...]))], axis=1)


def _passD_body(adj_ref, rhs_ref, b_ref, u_ref,
                w4ms_ref, wum1_ref, wum2_ref, bum_ref,
                wus1_ref, wus2_ref, bus_ref,
                rE_ref, mu_ref, lu_ref):
    t = _gcn_tile(adj_ref, rhs_ref, b_ref)
    ihh, gmu, gsu = t[:, :F], t[:, F:2 * F], t[:, 2 * F:]
    rE_ref[...] = _bf(_dot(_bf(ihh), w4ms_ref[...]))
    ub = _bf(u_ref[...])
    mu_ref[...] = (_dot(_bf(gmu), wum1_ref[...]) + _dot(ub, wum2_ref[...])
                   + bum_ref[...])
    lu_ref[...] = (_dot(_bf(gsu), wus1_ref[...]) + _dot(ub, wus2_ref[...])
                   + bus_ref[...])


def _passE_body(adj_ref, rhs_ref, b_ref, v_ref,
                wim1_ref, wim2_ref, bim_ref,
                wis1_ref, wis2_ref, bis_ref,
                mi_ref, li_ref):
    t = _gcn_tile(adj_ref, rhs_ref, b_ref)
    gmi, gsi = t[:, :F], t[:, F:]
    vb = _bf(v_ref[...])
    mi_ref[...] = (_dot(_bf(gmi), wim1_ref[...]) + _dot(vb, wim2_ref[...])
                   + bim_ref[...])
    li_ref[...] = (_dot(_bf(gsi), wis1_ref[...]) + _dot(vb, wis2_ref[...])
                   + bis_ref[...])


def kernel(ufea, vfea, UV_adj, VU_adj, params):
    p = params

    def wcat(*names):
        return _bf(jnp.concatenate([p[n] for n in names], axis=1))

    def bcat(*names):
        return jnp.concatenate([p[n] for n in names])[None, :]

    rA = _proj(ufea, _bf(p['l0_gc1_W']))

    vu_bf, rB = _pass(
        _passA_body, VU_adj, rA, p['l0_gc1_b'][None, :],
        [vfea], [_bf(p['l0_gc2_W']), _bf(p['l0_gc3_W'])],
        [N, 2 * F], [jnp.bfloat16, jnp.bfloat16])

    rC, u = _pass(
        _passB_body, UV_adj, rB, bcat('l0_gc2_b', 'l0_gc3_b'),
        [ufea],
        [_bf(p['l0_gc4_W']), _bf(p['ll_gc1_W']),
         _bf(p['l0_uu_W'][:F]), _bf(p['l0_uu_W'][F:]), p['l0_uu_b'][None, :]],
        [2 * F, F], [jnp.bfloat16, jnp.float32])

    rD, v = _pass(
        _passC_body, vu_bf, rC, bcat('l0_gc4_b', 'll_gc1_b'),
        [vfea],
        [_bf(p['ll_gc2_W']), wcat('ll_gc3m_W', 'll_gc3s_W'),
         _bf(p['l0_iu_W'][:F]), _bf(p['l0_iu_W'][F:]), p['l0_iu_b'][None, :]],
        [3 * F, F], [jnp.bfloat16, jnp.float32])

    rE, mean_u, logstd_u = _pass(
        _passD_body, UV_adj, rD, bcat('ll_gc2_b', 'll_gc3m_b', 'll_gc3s_b'),
        [u],
        [wcat('ll_gc4m_W', 'll_gc4s_W'),
         _bf(p['ll_uum_W'][:F]), _bf(p['ll_uum_W'][F:]), p['ll_uum_b'][None, :],
         _bf(p['ll_uus_W'][:F]), _bf(p['ll_uus_W'][F:]), p['ll_uus_b'][None, :]],
        [2 * F, F, F], [jnp.bfloat16, jnp.float32, jnp.float32])

    mean_i, logstd_i = _pass(
        _passE_body, vu_bf, rE, bcat('ll_gc4m_b', 'll_gc4s_b'),
        [v],
        [_bf(p['ll_ium_W'][:F]), _bf(p['ll_ium_W'][F:]), p['ll_ium_b'][None, :],
         _bf(p['ll_ius_W'][:F]), _bf(p['ll_ius_W'][F:]), p['ll_ius_b'][None, :]],
        [F, F], [jnp.float32, jnp.float32])

    return (mean_u, mean_i, mean_u, mean_i, logstd_u, logstd_i)


# speedup vs baseline: 1.9490x; 1.0941x over previous
import jax
import jax.numpy as jnp
from jax.experimental import pallas as pl
from jax.experimental.pallas import tpu as pltpu

N = 10000
F = 128
ALPHA = 0.3

_BM = 400
_BM_SMALL = 1000


def _leaky(x):
    return jnp.where(x >= 0, x, ALPHA * x)


def _bf(x):
    return x.astype(jnp.bfloat16)


def _dot(a, b):
    return jnp.dot(a, b, preferred_element_type=jnp.float32)


def _proj_body(x_ref, w_ref, o_ref):
    o_ref[...] = _bf(_dot(_bf(x_ref[...]), w_ref[...]))


def _proj(x, w_bf):
    w = w_bf.shape[1]
    return pl.pallas_call(
        _proj_body,
        grid=(N // _BM_SMALL,),
        in_specs=[pl.BlockSpec((_BM_SMALL, F), lambda i: (i, 0)),
                  pl.BlockSpec((F, w), lambda i: (0, 0))],
        out_specs=pl.BlockSpec((_BM_SMALL, w), lambda i: (i, 0)),
        out_shape=jax.ShapeDtypeStruct((N, w), jnp.bfloat16),
    )(x, w_bf)


def _pass(body, adj, rhs, bias, row_ins, res_ins, out_w, out_dt):
    w = rhs.shape[1]
    in_specs = [pl.BlockSpec((_BM, N), lambda i: (i, 0)),
                pl.BlockSpec((N, w), lambda i: (0, 0)),
                pl.BlockSpec((1, w), lambda i: (0, 0))]
    for a in row_ins:
        in_specs.append(pl.BlockSpec((_BM, a.shape[1]), lambda i: (i, 0)))
    for a in res_ins:
        in_specs.append(pl.BlockSpec(
            tuple(a.shape), lambda i, n=len(a.shape): (0,) * n))
    out_specs = [pl.BlockSpec((_BM, ww), lambda i: (i, 0)) for ww in out_w]
    out_shape = [jax.ShapeDtypeStruct((N, ww), dt)
                 for ww, dt in zip(out_w, out_dt)]
    return pl.pallas_call(
        body,
        grid=(N // _BM,),
        in_specs=in_specs,
        out_specs=out_specs,
        out_shape=out_shape,
        compiler_params=pltpu.CompilerParams(
            dimension_semantics=("arbitrary",)),
    )(adj, rhs, bias, *row_ins, *res_ins)


def _gcn_tile(adj_ref, rhs_ref, b_ref):
    a = adj_ref[...]
    if a.dtype != jnp.bfloat16:
        a = _bf(a)
    return _leaky(_dot(a, rhs_ref[...]) + b_ref[...])


def _passA_body(adj_ref, rhs_ref, b_ref, vfea_ref, w2_ref, w3_ref,
                abf_ref, rB_ref):
    a = _bf(adj_ref[...])
    abf_ref[...] = a
    uh1 = _leaky(_dot(a, rhs_ref[...]) + b_ref[...])
    rB_ref[...] = jnp.concatenate(
        [_bf(_dot(_bf(vfea_ref[...]), w2_ref[...])),
         _bf(_dot(_bf(uh1), w3_ref[...]))], axis=1)


def _passB_body(adj_ref, rhs_ref, b_ref, ufea_ref,
                w4_ref, wll1_ref, wuu1_ref, wuu2_ref, buu_ref,
                rC_ref, u_ref):
    t = _gcn_tile(adj_ref, rhs_ref, b_ref)
    ih1, uh2 = t[:, :F], t[:, F:]
    u = jnp.maximum(_dot(_bf(uh2), wuu1_ref[...])
                    + _dot(_bf(ufea_ref[...]), wuu2_ref[...])
                    + buu_ref[...], 0.0)
    u_ref[...] = u
    rC_ref[...] = jnp.concatenate(
        [_bf(_dot(_bf(ih1), w4_ref[...])),
         _bf(_dot(_bf(u), wll1_ref[...]))], axis=1)


def _passC_body(adj_ref, rhs_ref, b_ref, vfea_ref,
                wll2_ref, w3ms_ref, wiu1_ref, wiu2_ref, biu_ref,
                rD_ref, v_ref):
    t = _gcn_tile(adj_ref, rhs_ref, b_ref)
    ih2, uhh = t[:, :F], t[:, F:]
    v = jnp.maximum(_dot(_bf(ih2), wiu1_ref[...])
                    + _dot(_bf(vfea_ref[...]), wiu2_ref[...])
                    + biu_ref[...], 0.0)
    v_ref[...] = v
    rD_ref[...] = jnp.concatenate(
        [_bf(_dot(_bf(v), wll2_ref[...])),
         _bf(_dot(_bf(uhh), w3ms_ref[...]))], axis=1)


def _passD_body(adj_ref, rhs_ref, b_ref, u_ref,
                w4ms_ref, wum1_ref, wum2_ref, bum_ref,
                wus1_ref, wus2_ref, bus_ref,
                rE_ref, mu_ref, lu_ref):
    t = _gcn_tile(adj_ref, rhs_ref, b_ref)
    ihh, gmu, gsu = t[:, :F], t[:, F:2 * F], t[:, 2 * F:]
    rE_ref[...] = _bf(_dot(_bf(ihh), w4ms_ref[...]))
    ub = _bf(u_ref[...])
    mu_ref[...] = (_dot(_bf(gmu), wum1_ref[...]) + _dot(ub, wum2_ref[...])
                   + bum_ref[...])
    lu_ref[...] = (_dot(_bf(gsu), wus1_ref[...]) + _dot(ub, wus2_ref[...])
                   + bus_ref[...])


def _passE_body(adj_ref, rhs_ref, b_ref, v_ref,
                wim1_ref, wim2_ref, bim_ref,
                wis1_ref, wis2_ref, bis_ref,
                mi_ref, li_ref):
    t = _gcn_tile(adj_ref, rhs_ref, b_ref)
    gmi, gsi = t[:, :F], t[:, F:]
    vb = _bf(v_ref[...])
    mi_ref[...] = (_dot(_bf(gmi), wim1_ref[...]) + _dot(vb, wim2_ref[...])
                   + bim_ref[...])
    li_ref[...] = (_dot(_bf(gsi), wis1_ref[...]) + _dot(vb, wis2_ref[...])
                   + bis_ref[...])


def kernel(ufea, vfea, UV_adj, VU_adj, params):
    p = params

    def wcat(*names):
        return _bf(jnp.concatenate([p[n] for n in names], axis=1))

    def bcat(*names):
        return jnp.concatenate([p[n] for n in names])[None, :]

    rA = _proj(ufea, _bf(p['l0_gc1_W']))

    vu_bf, rB = _pass(
        _passA_body, VU_adj, rA, p['l0_gc1_b'][None, :],
        [vfea], [_bf(p['l0_gc2_W']), _bf(p['l0_gc3_W'])],
        [N, 2 * F], [jnp.bfloat16, jnp.bfloat16])

    rC, u = _pass(
        _passB_body, UV_adj, rB, bcat('l0_gc2_b', 'l0_gc3_b'),
        [ufea],
        [_bf(p['l0_gc4_W']), _bf(p['ll_gc1_W']),
         _bf(p['l0_uu_W'][:F]), _bf(p['l0_uu_W'][F:]), p['l0_uu_b'][None, :]],
        [2 * F, F], [jnp.bfloat16, jnp.float32])

    rD, v = _pass(
        _passC_body, vu_bf, rC, bcat('l0_gc4_b', 'll_gc1_b'),
        [vfea],
        [_bf(p['ll_gc2_W']), wcat('ll_gc3m_W', 'll_gc3s_W'),
         _bf(p['l0_iu_W'][:F]), _bf(p['l0_iu_W'][F:]), p['l0_iu_b'][None, :]],
        [3 * F, F], [jnp.bfloat16, jnp.float32])

    rE, mean_u, logstd_u = _pass(
        _passD_body, UV_adj, rD, bcat('ll_gc2_b', 'll_gc3m_b', 'll_gc3s_b'),
        [u],
        [wcat('ll_gc4m_W', 'll_gc4s_W'),
         _bf(p['ll_uum_W'][:F]), _bf(p['ll_uum_W'][F:]), p['ll_uum_b'][None, :],
         _bf(p['ll_uus_W'][:F]), _bf(p['ll_uus_W'][F:]), p['ll_uus_b'][None, :]],
        [2 * F, F, F], [jnp.bfloat16, jnp.float32, jnp.float32])

    mean_i, logstd_i = _pass(
        _passE_body, vu_bf, rE, bcat('ll_gc4m_b', 'll_gc4s_b'),
        [v],
        [_bf(p['ll_ium_W'][:F]), _bf(p['ll_ium_W'][F:]), p['ll_ium_b'][None, :],
         _bf(p['ll_ius_W'][:F]), _bf(p['ll_ius_W'][F:]), p['ll_ius_b'][None, :]],
        [F, F], [jnp.float32, jnp.float32])

    return (mean_u, mean_i, mean_u, mean_i, logstd_u, logstd_i)


# speedup vs baseline: 1.9913x; 1.0217x over previous
import jax
import jax.numpy as jnp
from jax.experimental import pallas as pl
from jax.experimental.pallas import tpu as pltpu

N = 10000
F = 128
ALPHA = 0.3

_BM = 400
_BM_SMALL = 1000


def _leaky(x):
    return jnp.where(x >= 0, x, ALPHA * x)


def _bf(x):
    return x.astype(jnp.bfloat16)


def _dot(a, b):
    return jnp.dot(a, b, preferred_element_type=jnp.float32)


def _proj_body(x_ref, w_ref, o_ref):
    o_ref[...] = _bf(_dot(_bf(x_ref[...]), w_ref[...]))


def _proj(x, w_bf):
    w = w_bf.shape[1]
    return pl.pallas_call(
        _proj_body,
        grid=(N // _BM_SMALL,),
        in_specs=[pl.BlockSpec((_BM_SMALL, F), lambda i: (i, 0)),
                  pl.BlockSpec((F, w), lambda i: (0, 0))],
        out_specs=pl.BlockSpec((_BM_SMALL, w), lambda i: (i, 0)),
        out_shape=jax.ShapeDtypeStruct((N, w), jnp.bfloat16),
    )(x, w_bf)


def _pass(body, adj, rhs, bias, row_ins, res_ins, out_w, out_dt, bm=_BM):
    w = rhs.shape[1]
    in_specs = [pl.BlockSpec((bm, N), lambda i: (i, 0)),
                pl.BlockSpec((N, w), lambda i: (0, 0)),
                pl.BlockSpec((1, w), lambda i: (0, 0))]
    for a in row_ins:
        in_specs.append(pl.BlockSpec((bm, a.shape[1]), lambda i: (i, 0)))
    for a in res_ins:
        in_specs.append(pl.BlockSpec(
            tuple(a.shape), lambda i, n=len(a.shape): (0,) * n))
    out_specs = [pl.BlockSpec((bm, ww), lambda i: (i, 0)) for ww in out_w]
    out_shape = [jax.ShapeDtypeStruct((N, ww), dt)
                 for ww, dt in zip(out_w, out_dt)]
    return pl.pallas_call(
        body,
        grid=(N // bm,),
        in_specs=in_specs,
        out_specs=out_specs,
        out_shape=out_shape,
        compiler_params=pltpu.CompilerParams(
            dimension_semantics=("arbitrary",)),
    )(adj, rhs, bias, *row_ins, *res_ins)


def _gcn_tile(adj_ref, rhs_ref, b_ref):
    a = adj_ref[...]
    if a.dtype != jnp.bfloat16:
        a = _bf(a)
    return _leaky(_dot(a, rhs_ref[...]) + b_ref[...])


def _passA_body(adj_ref, rhs_ref, b_ref, vfea_ref, w2_ref, w3_ref,
                abf_ref, rB_ref):
    a = _bf(adj_ref[...])
    abf_ref[...] = a
    uh1 = _leaky(_dot(a, rhs_ref[...]) + b_ref[...])
    rB_ref[...] = jnp.concatenate(
        [_bf(_dot(_bf(vfea_ref[...]), w2_ref[...])),
         _bf(_dot(_bf(uh1), w3_ref[...]))], axis=1)


def _passB_body(adj_ref, rhs_ref, b_ref, ufea_ref,
                w4_ref, wll1_ref, wuu1_ref, wuu2_ref, buu_ref,
                rC_ref, u_ref):
    t = _gcn_tile(adj_ref, rhs_ref, b_ref)
    ih1, uh2 = t[:, :F], t[:, F:]
    u = jnp.maximum(_dot(_bf(uh2), wuu1_ref[...])
                    + _dot(_bf(ufea_ref[...]), wuu2_ref[...])
                    + buu_ref[...], 0.0)
    u_ref[...] = u
    rC_ref[...] = jnp.concatenate(
        [_bf(_dot(_bf(ih1), w4_ref[...])),
         _bf(_dot(_bf(u), wll1_ref[...]))], axis=1)


def _passC_body(adj_ref, rhs_ref, b_ref, vfea_ref,
                wll2_ref, w3ms_ref, wiu1_ref, wiu2_ref, biu_ref,
                rD_ref, v_ref):
    t = _gcn_tile(adj_ref, rhs_ref, b_ref)
    ih2, uhh = t[:, :F], t[:, F:]
    v = jnp.maximum(_dot(_bf(ih2), wiu1_ref[...])
                    + _dot(_bf(vfea_ref[...]), wiu2_ref[...])
                    + biu_ref[...], 0.0)
    v_ref[...] = v
    rD_ref[...] = jnp.concatenate(
        [_bf(_dot(_bf(v), wll2_ref[...])),
         _bf(_dot(_bf(uhh), w3ms_ref[...]))], axis=1)


def _passD_body(adj_ref, rhs_ref, b_ref, u_ref,
                w4ms_ref, wum1_ref, wum2_ref, bum_ref,
                wus1_ref, wus2_ref, bus_ref,
                rE_ref, mu_ref, lu_ref):
    t = _gcn_tile(adj_ref, rhs_ref, b_ref)
    ihh, gmu, gsu = t[:, :F], t[:, F:2 * F], t[:, 2 * F:]
    rE_ref[...] = _bf(_dot(_bf(ihh), w4ms_ref[...]))
    ub = _bf(u_ref[...])
    mu_ref[...] = (_dot(_bf(gmu), wum1_ref[...]) + _dot(ub, wum2_ref[...])
                   + bum_ref[...])
    lu_ref[...] = (_dot(_bf(gsu), wus1_ref[...]) + _dot(ub, wus2_ref[...])
                   + bus_ref[...])


def _passE_body(adj_ref, rhs_ref, b_ref, v_ref,
                wim1_ref, wim2_ref, bim_ref,
                wis1_ref, wis2_ref, bis_ref,
                mi_ref, li_ref):
    t = _gcn_tile(adj_ref, rhs_ref, b_ref)
    gmi, gsi = t[:, :F], t[:, F:]
    vb = _bf(v_ref[...])
    mi_ref[...] = (_dot(_bf(gmi), wim1_ref[...]) + _dot(vb, wim2_ref[...])
                   + bim_ref[...])
    li_ref[...] = (_dot(_bf(gsi), wis1_ref[...]) + _dot(vb, wis2_ref[...])
                   + bis_ref[...])


def kernel(ufea, vfea, UV_adj, VU_adj, params):
    p = params

    def wcat(*names):
        return _bf(jnp.concatenate([p[n] for n in names], axis=1))

    def bcat(*names):
        return jnp.concatenate([p[n] for n in names])[None, :]

    rA = _proj(ufea, _bf(p['l0_gc1_W']))

    vu_bf, rB = _pass(
        _passA_body, VU_adj, rA, p['l0_gc1_b'][None, :],
        [vfea], [_bf(p['l0_gc2_W']), _bf(p['l0_gc3_W'])],
        [N, 2 * F], [jnp.bfloat16, jnp.bfloat16])

    rC, u = _pass(
        _passB_body, UV_adj, rB, bcat('l0_gc2_b', 'l0_gc3_b'),
        [ufea],
        [_bf(p['l0_gc4_W']), _bf(p['ll_gc1_W']),
         _bf(p['l0_uu_W'][:F]), _bf(p['l0_uu_W'][F:]), p['l0_uu_b'][None, :]],
        [2 * F, F], [jnp.bfloat16, jnp.float32])

    rD, v = _pass(
        _passC_body, vu_bf, rC, bcat('l0_gc4_b', 'll_gc1_b'),
        [vfea],
        [_bf(p['ll_gc2_W']), wcat('ll_gc3m_W', 'll_gc3s_W'),
         _bf(p['l0_iu_W'][:F]), _bf(p['l0_iu_W'][F:]), p['l0_iu_b'][None, :]],
        [3 * F, F], [jnp.bfloat16, jnp.float32], bm=1000)

    rE, mean_u, logstd_u = _pass(
        _passD_body, UV_adj, rD, bcat('ll_gc2_b', 'll_gc3m_b', 'll_gc3s_b'),
        [u],
        [wcat('ll_gc4m_W', 'll_gc4s_W'),
         _bf(p['ll_uum_W'][:F]), _bf(p['ll_uum_W'][F:]), p['ll_uum_b'][None, :],
         _bf(p['ll_uus_W'][:F]), _bf(p['ll_uus_W'][F:]), p['ll_uus_b'][None, :]],
        [2 * F, F, F], [jnp.bfloat16, jnp.float32, jnp.float32])

    mean_i, logstd_i = _pass(
        _passE_body, vu_bf, rE, bcat('ll_gc4m_b', 'll_gc4s_b'),
        [v],
        [_bf(p['ll_ium_W'][:F]), _bf(p['ll_ium_W'][F:]), p['ll_ium_b'][None, :],
         _bf(p['ll_ius_W'][:F]), _bf(p['ll_ius_W'][F:]), p['ll_ius_b'][None, :]],
        [F, F], [jnp.float32, jnp.float32], bm=1000)

    return (mean_u, mean_i, mean_u, mean_i, logstd_u, logstd_i)
